# baseline (speedup 1.0000x reference)
import jax
import jax.numpy as jnp
from jax import lax
from jax.experimental import pallas as pl
from jax.experimental.pallas import tpu as pltpu
from jax.experimental.pallas import tpu_sc as plsc

NC = 2
NS = 16
L = 16

BSZ = 128
BEAMS = 4
VOCAB = 100000
VK = 8
ROWS_PER_TILE = BSZ // NC // NS
NCHUNK = VOCAB // L
CPB = 50
NBLK = NCHUNK // CPB
PIECE_V = 80
PIECE_W = PIECE_V * BSZ
PPB = 10
VB_CAP = 12
CAND_CAP = 32
CMC = BSZ // NC
GPS = CMC // L

NEG_INF = float("-inf")
BIG_I32 = 2**31 - 1


def _topk_body(step_hbm, lp_hbm, scores_hbm,
               out_val_hbm, out_idx_hbm, out_beam_hbm,
               pbuf, cmstage, bmstage, bmloc, cmblk, blkids, cidlist,
               candidx, canddata, vidbuf, st_val, st_idx, st_beam,
               step_v, scores_v, cm_sh, bm_sh, cnt_smem, sem, sem2):
    c = lax.axis_index("c")
    s = lax.axis_index("s")
    lane = lax.iota(jnp.int32, L)
    minus_inf = jnp.full((L,), NEG_INF, jnp.float32)
    plus_inf = jnp.full((L,), float("inf"), jnp.float32)
    big_vec = jnp.full((L,), BIG_I32, jnp.int32)
    zero_i = jnp.zeros((L,), jnp.int32)

    pltpu.sync_copy(step_hbm, step_v)
    pltpu.sync_copy(scores_hbm, scores_v)
    ssum = jnp.zeros((L,), jnp.float32)
    for i in range(BSZ * BEAMS // L):
        ssum = ssum + scores_v[pl.ds(i * L, L)]
    mean = jnp.sum(ssum) * (1.0 / (BSZ * BEAMS))
    stepf = jnp.max(step_v[...].astype(jnp.float32))
    shift = stepf * mean

    sb = jnp.where(s < 13, 8 * s, 104 + 7 * (s - 13))
    nb = jnp.where(s < 13, 8, 7)

    with jax.named_scope("phaseA"):
        def piece_body(p, bmacc):
            vstart = sb * (CPB * L) + p * PIECE_V
            pltpu.sync_copy(lp_hbm.at[pl.ds(vstart * BSZ, PIECE_W)], pbuf)
            first = (s == 0) & (p == 0)
            new_acc = []
            for g in range(GPS):
                bacc_g = bmacc[g]
                for c5 in range(5):
                    acc = minus_inf
                    for j in range(L):
                        x = pbuf[pl.ds(
                            (c5 * L + j) * BSZ + (c * GPS + g) * L, L)]
                        if c5 == 0 and j == 0:
                            x = jnp.where(first, minus_inf, x)
                        acc = jnp.maximum(acc, x)
                    cmstage[pl.ds(c5 * CMC + g * L, L)] = acc
                    bacc_g = jnp.maximum(bacc_g, acc)
                new_acc.append(bacc_g)
            pltpu.sync_copy(
                cmstage, cm_sh.at[pl.ds((sb * CPB + p * 5) * CMC, 5 * CMC)])

            @pl.when((p + 1) % PPB == 0)
            def _flush_bm():
                for g in range(GPS):
                    bmstage[pl.ds(g * L, L)] = new_acc[g]
                pltpu.sync_copy(
                    bmstage, bm_sh.at[pl.ds((sb + p // PPB) * CMC, CMC)])

            done = (p + 1) % PPB == 0
            return tuple(jnp.where(done, minus_inf, a) for a in new_acc)

        lax.fori_loop(0, nb * PPB, piece_body, (minus_inf,) * GPS)

    plsc.subcore_barrier()

    pltpu.sync_copy(bm_sh, bmloc.at[pl.ds(0, NBLK * CMC)])

    def row_body(rr, row_carry):
        rloc = s * ROWS_PER_TILE + rr
        r = c * (NS * ROWS_PER_TILE) + rloc

        bmo = []
        for k in range(8):
            bv = plsc.load_gather(bmloc, [(lane + k * L) * CMC + rloc])
            if k == 7:
                bv = jnp.where(lane + 112 < NBLK, bv, minus_inf)
            bmo.append(bv)

        wk = list(bmo)
        t = jnp.float32(0)
        for it in range(VK):
            mx = wk[0]
            for k in range(1, 8):
                mx = jnp.maximum(mx, wk[k])
            m = jnp.max(mx)
            m_vec = jnp.full((L,), m, jnp.float32)
            wk = [jnp.where(w == m_vec, minus_inf, w) for w in wk]
            t = m
        t_vec = jnp.full((L,), t, jnp.float32)

        cnt_smem[0] = 0
        for k in range(8):
            hits = bmo[k] >= t_vec
            s0 = cnt_smem[0]

            @pl.when(s0 <= VB_CAP)
            def _stb(hits=hits, k=k, s0=s0):
                plsc.store_compressed(
                    blkids.at[pl.ds(s0, L)], lane + k * L, mask=hits)
                cnt_smem[0] = s0 + jnp.max(
                    plsc.all_reduce_population_count(hits))
        nvb = cnt_smem[0]
        bv16 = blkids[pl.ds(0, L)]

        for j in range(VB_CAP):
            b = bv16[j]

            @pl.when(j < nvb)
            def _fire(b=b, j=j):
                pltpu.async_copy(
                    cm_sh.at[pl.ds(b * CPB * CMC, CPB * CMC)],
                    cmblk.at[pl.ds(j * CPB * CMC, CPB * CMC)], sem)
        for j in range(VB_CAP):
            @pl.when(j < nvb)
            def _drain(j=j):
                pltpu.make_async_copy(
                    lp_hbm.at[pl.ds(0, CPB * CMC)],
                    cmblk.at[pl.ds(j * CPB * CMC, CPB * CMC)], sem).wait()

        cnt_smem[1] = 0
        for j in range(VB_CAP):
            b = bv16[j]

            @pl.when(j < nvb)
            def _cand(b=b, j=j):
                for k4 in range(4):
                    cloc = lane + k4 * L
                    col = plsc.load_gather(
                        cmblk, [j * CPB * CMC + cloc * CMC + rloc])
                    h = (col >= t_vec) & (cloc < CPB)
                    s2 = cnt_smem[1]

                    @pl.when(s2 <= CAND_CAP - L)
                    def _stc(h=h, b=b, cloc=cloc, s2=s2):
                        plsc.store_compressed(
                            cidlist.at[pl.ds(s2, L)], b * CPB + cloc, mask=h)
                        cnt_smem[1] = s2 + jnp.max(
                            plsc.all_reduce_population_count(h))
        ncand = cnt_smem[1]
        cidlist[pl.ds(ncand, L)] = zero_i

        ncand_vec = jnp.full((L,), ncand, jnp.int32)
        for q in range(CAND_CAP // L):
            cidv = cidlist[pl.ds(q * L, L)]
            cidv = jnp.where(lane + q * L < ncand_vec, cidv, 0)
            for j16 in range(L):
                v = cidv * L + j16
                addr = (v // 8) * (8 * BSZ) + (v % 8) * BSZ + r
                w = q * L + j16
                candidx[w // 8, pl.ds((w % 8) * L, L)] = addr
                vidbuf[pl.ds(w * L, L)] = v
        for q2 in range(CAND_CAP * L // BSZ):
            pltpu.async_copy(
                lp_hbm.at[candidx.at[q2]], canddata.at[q2], sem2)
        for q2 in range(CAND_CAP * L // BSZ):
            pltpu.make_async_copy(
                lp_hbm.at[pl.ds(0, BSZ)], canddata.at[q2], sem2).wait()

        for w in range(CAND_CAP):
            x = canddata[w // 8, pl.ds((w % 8) * L, L)]
            vid = vidbuf[pl.ds(w * L, L)]
            canddata[w // 8, pl.ds((w % 8) * L, L)] = jnp.where(
                vid == 0, minus_inf, x)

        ovv = minus_inf
        oiv = zero_i
        for k in range(VK):
            macc = minus_inf
            for w in range(CAND_CAP):
                macc = jnp.maximum(
                    macc, canddata[w // 8, pl.ds((w % 8) * L, L)])
            m = jnp.max(macc)
            m_vec = jnp.full((L,), m, jnp.float32)

            iacc = big_vec
            for w in range(CAND_CAP):
                x = canddata[w // 8, pl.ds((w % 8) * L, L)]
                vid = vidbuf[pl.ds(w * L, L)]
                iacc = jnp.minimum(iacc, jnp.where(x == m_vec, vid, big_vec))
            ci = jnp.min(iacc)
            ci_vec = jnp.full((L,), ci, jnp.int32)

            for w in range(CAND_CAP):
                x = canddata[w // 8, pl.ds((w % 8) * L, L)]
                vid = vidbuf[pl.ds(w * L, L)]
                canddata[w // 8, pl.ds((w % 8) * L, L)] = jnp.where(
                    vid == ci_vec, minus_inf, x)

            ovv = jnp.where(lane == k, m_vec, ovv)
            oiv = jnp.where(lane == k, ci_vec, oiv)

        sh_vec = jnp.full((L,), shift, jnp.float32)
        st_val[pl.ds(rr * L, L)] = ovv + sh_vec
        st_idx[pl.ds(rr * L, L)] = oiv
        st_beam[pl.ds(rr * L, L)] = zero_i
        return row_carry

    with jax.named_scope("phaseB"):
        lax.fori_loop(0, ROWS_PER_TILE, row_body, 0)

    base = (c * (NS * ROWS_PER_TILE) + s * ROWS_PER_TILE) * L
    pltpu.sync_copy(st_val, out_val_hbm.at[pl.ds(base, ROWS_PER_TILE * L)])
    pltpu.sync_copy(st_idx, out_idx_hbm.at[pl.ds(base, ROWS_PER_TILE * L)])
    pltpu.sync_copy(st_beam, out_beam_hbm.at[pl.ds(base, ROWS_PER_TILE * L)])


@jax.jit
def _sc_topk(step_v, lp_flat, scores_flat):
    mesh = plsc.VectorSubcoreMesh(
        core_axis_name="c", subcore_axis_name="s",
        num_cores=NC, num_subcores=NS)
    fn = pl.kernel(
        _topk_body,
        out_type=(
            jax.ShapeDtypeStruct((BSZ * L,), jnp.float32),
            jax.ShapeDtypeStruct((BSZ * L,), jnp.int32),
            jax.ShapeDtypeStruct((BSZ * L,), jnp.int32),
        ),
        mesh=mesh,
        compiler_params=pltpu.CompilerParams(needs_layout_passes=False),
        scratch_types=[
            pltpu.VMEM((PIECE_W,), jnp.float32),
            pltpu.VMEM((5 * CMC,), jnp.float32),
            pltpu.VMEM((CMC,), jnp.float32),
            pltpu.VMEM((BSZ * CMC,), jnp.float32),
            pltpu.VMEM((VB_CAP * CPB * CMC + CMC * L,),
                       jnp.float32),
            pltpu.VMEM((VB_CAP + 2 * L,), jnp.int32),
            pltpu.VMEM((CAND_CAP + L,), jnp.int32),
            pltpu.VMEM((CAND_CAP * L // BSZ, BSZ), jnp.int32),
            pltpu.VMEM((CAND_CAP * L // BSZ, BSZ), jnp.float32),
            pltpu.VMEM((CAND_CAP * L,), jnp.int32),
            pltpu.VMEM((ROWS_PER_TILE * L,), jnp.float32),
            pltpu.VMEM((ROWS_PER_TILE * L,), jnp.int32),
            pltpu.VMEM((ROWS_PER_TILE * L,), jnp.int32),
            pltpu.VMEM((L,), jnp.int32),
            pltpu.VMEM((BSZ * BEAMS,), jnp.float32),
            pltpu.VMEM_SHARED((NCHUNK * CMC,), jnp.float32),
            pltpu.VMEM_SHARED((NBLK * CMC,), jnp.float32),
            pltpu.SMEM((2,), jnp.int32),
            pltpu.SemaphoreType.DMA,
            pltpu.SemaphoreType.DMA,
        ],
    )
    return fn(step_v, lp_flat, scores_flat)


def kernel(step, lprobs, scores):
    step_v = jnp.broadcast_to(
        jnp.asarray(step, jnp.int32).reshape(()), (L,))
    lp_flat = jnp.transpose(lprobs, (1, 2, 0)).reshape(
        BEAMS, VOCAB // 8, 8, BSZ).reshape(-1)
    sc, ix, bm = _sc_topk(step_v, lp_flat, scores.reshape(-1))
    return (sc.reshape(BSZ, L)[:, :VK], ix.reshape(BSZ, L)[:, :VK],
            bm.reshape(BSZ, L)[:, :VK])


# speedup vs baseline: 1.5242x; 1.5242x over previous
import jax
import jax.numpy as jnp
from jax import lax
from jax.experimental import pallas as pl
from jax.experimental.pallas import tpu as pltpu
from jax.experimental.pallas import tpu_sc as plsc

NC = 2
NS = 16
L = 16

BSZ = 128
BEAMS = 4
VOCAB = 100000
VK = 8
ROWS_PER_TILE = BSZ // NC // NS
NCHUNK = VOCAB // L
CPB = 50
NBLK = NCHUNK // CPB
PIECE_V = 80
PIECE_W = PIECE_V * BSZ
PPB = 10
VB_CAP = 12
CAND_CAP = 32
CMC = BSZ // NC
GPS = CMC // L

NEG_INF = float("-inf")
BIG_I32 = 2**31 - 1


def _topk_body(step_hbm, lp_hbm, scores_hbm,
               out_val_hbm, out_idx_hbm, out_beam_hbm,
               pbuf, pbuf2, cmstage, bmstage, bmloc, cmblk, blkids, cidlist,
               candidx, canddata, vidbuf, st_val, st_idx, st_beam,
               step_v, scores_v, cm_sh, bm_sh, cnt_smem, sem, sem2):
    c = lax.axis_index("c")
    s = lax.axis_index("s")
    lane = lax.iota(jnp.int32, L)
    minus_inf = jnp.full((L,), NEG_INF, jnp.float32)
    plus_inf = jnp.full((L,), float("inf"), jnp.float32)
    big_vec = jnp.full((L,), BIG_I32, jnp.int32)
    zero_i = jnp.zeros((L,), jnp.int32)

    pltpu.sync_copy(step_hbm, step_v)
    pltpu.sync_copy(scores_hbm, scores_v)
    ssum = jnp.zeros((L,), jnp.float32)
    for i in range(BSZ * BEAMS // L):
        ssum = ssum + scores_v[pl.ds(i * L, L)]
    mean = jnp.sum(ssum) * (1.0 / (BSZ * BEAMS))
    stepf = jnp.max(step_v[...].astype(jnp.float32))
    shift = stepf * mean

    sb = jnp.where(s < 13, 8 * s, 104 + 7 * (s - 13))
    nb = jnp.where(s < 13, 8, 7)

    np_ = nb * PPB

    def fire(p, buf, psem):
        vstart = sb * (CPB * L) + p * PIECE_V
        pltpu.async_copy(lp_hbm.at[pl.ds(vstart * BSZ, PIECE_W)], buf, psem)

    def wait_piece(buf, psem):
        pltpu.make_async_copy(lp_hbm.at[pl.ds(0, PIECE_W)], buf, psem).wait()

    def compute_piece(buf, p, bmacc):
        first = (s == 0) & (p == 0)
        new_acc = []
        for g in range(GPS):
            bacc_g = bmacc[g]
            for c5 in range(5):
                acc = minus_inf
                for j in range(L):
                    x = buf[pl.ds(
                        (c5 * L + j) * BSZ + (c * GPS + g) * L, L)]
                    if c5 == 0 and j == 0:
                        x = jnp.where(first, minus_inf, x)
                    acc = jnp.maximum(acc, x)
                cmstage[pl.ds(c5 * CMC + g * L, L)] = acc
                bacc_g = jnp.maximum(bacc_g, acc)
            new_acc.append(bacc_g)
        pltpu.sync_copy(
            cmstage, cm_sh.at[pl.ds((sb * CPB + p * 5) * CMC, 5 * CMC)])
        return new_acc

    with jax.named_scope("phaseA"):
        fire(0, pbuf, sem)
        fire(1, pbuf2, sem2)

        def pair_body(q, bmacc):
            p0 = 2 * q
            p1 = 2 * q + 1
            wait_piece(pbuf, sem)

            @pl.when(p0 + 2 < np_)
            def _f0():
                fire(p0 + 2, pbuf, sem)
            bmacc = tuple(compute_piece(pbuf, p0, bmacc))
            wait_piece(pbuf2, sem2)

            @pl.when(p1 + 2 < np_)
            def _f1():
                fire(p1 + 2, pbuf2, sem2)
            new_acc = compute_piece(pbuf2, p1, bmacc)

            @pl.when((p1 + 1) % PPB == 0)
            def _flush_bm():
                for g in range(GPS):
                    bmstage[pl.ds(g * L, L)] = new_acc[g]
                pltpu.sync_copy(
                    bmstage, bm_sh.at[pl.ds((sb + p1 // PPB) * CMC, CMC)])

            done = (p1 + 1) % PPB == 0
            return tuple(jnp.where(done, minus_inf, a) for a in new_acc)

        lax.fori_loop(0, np_ // 2, pair_body, (minus_inf,) * GPS)

    plsc.subcore_barrier()

    pltpu.sync_copy(bm_sh, bmloc.at[pl.ds(0, NBLK * CMC)])

    def row_body(rr, row_carry):
        rloc = s * ROWS_PER_TILE + rr
        r = c * (NS * ROWS_PER_TILE) + rloc

        bmo = []
        for k in range(8):
            bv = plsc.load_gather(bmloc, [(lane + k * L) * CMC + rloc])
            if k == 7:
                bv = jnp.where(lane + 112 < NBLK, bv, minus_inf)
            bmo.append(bv)

        wk = list(bmo)
        t = jnp.float32(0)
        for it in range(VK):
            mx = wk[0]
            for k in range(1, 8):
                mx = jnp.maximum(mx, wk[k])
            m = jnp.max(mx)
            m_vec = jnp.full((L,), m, jnp.float32)
            wk = [jnp.where(w == m_vec, minus_inf, w) for w in wk]
            t = m
        t_vec = jnp.full((L,), t, jnp.float32)

        cnt_smem[0] = 0
        for k in range(8):
            hits = bmo[k] >= t_vec
            s0 = cnt_smem[0]

            @pl.when(s0 <= VB_CAP)
            def _stb(hits=hits, k=k, s0=s0):
                plsc.store_compressed(
                    blkids.at[pl.ds(s0, L)], lane + k * L, mask=hits)
                cnt_smem[0] = s0 + jnp.max(
                    plsc.all_reduce_population_count(hits))
        nvb = cnt_smem[0]
        bv16 = blkids[pl.ds(0, L)]

        for j in range(VB_CAP):
            b = bv16[j]

            @pl.when(j < nvb)
            def _fire(b=b, j=j):
                pltpu.async_copy(
                    cm_sh.at[pl.ds(b * CPB * CMC, CPB * CMC)],
                    cmblk.at[pl.ds(j * CPB * CMC, CPB * CMC)], sem)
        for j in range(VB_CAP):
            @pl.when(j < nvb)
            def _drain(j=j):
                pltpu.make_async_copy(
                    lp_hbm.at[pl.ds(0, CPB * CMC)],
                    cmblk.at[pl.ds(j * CPB * CMC, CPB * CMC)], sem).wait()

        cnt_smem[1] = 0
        for j in range(VB_CAP):
            b = bv16[j]

            @pl.when(j < nvb)
            def _cand(b=b, j=j):
                for k4 in range(4):
                    cloc = lane + k4 * L
                    col = plsc.load_gather(
                        cmblk, [j * CPB * CMC + cloc * CMC + rloc])
                    h = (col >= t_vec) & (cloc < CPB)
                    s2 = cnt_smem[1]

                    @pl.when(s2 <= CAND_CAP - L)
                    def _stc(h=h, b=b, cloc=cloc, s2=s2):
                        plsc.store_compressed(
                            cidlist.at[pl.ds(s2, L)], b * CPB + cloc, mask=h)
                        cnt_smem[1] = s2 + jnp.max(
                            plsc.all_reduce_population_count(h))
        ncand = cnt_smem[1]
        cidlist[pl.ds(ncand, L)] = zero_i

        ncand_vec = jnp.full((L,), ncand, jnp.int32)
        for q in range(CAND_CAP // L):
            cidv = cidlist[pl.ds(q * L, L)]
            cidv = jnp.where(lane + q * L < ncand_vec, cidv, 0)
            for j16 in range(L):
                v = cidv * L + j16
                addr = (v // 8) * (8 * BSZ) + (v % 8) * BSZ + r
                w = q * L + j16
                candidx[w // 8, pl.ds((w % 8) * L, L)] = addr
                vidbuf[pl.ds(w * L, L)] = v
        for q2 in range(CAND_CAP * L // BSZ):
            pltpu.async_copy(
                lp_hbm.at[candidx.at[q2]], canddata.at[q2], sem2)
        for q2 in range(CAND_CAP * L // BSZ):
            pltpu.make_async_copy(
                lp_hbm.at[pl.ds(0, BSZ)], canddata.at[q2], sem2).wait()

        for w in range(CAND_CAP):
            x = canddata[w // 8, pl.ds((w % 8) * L, L)]
            vid = vidbuf[pl.ds(w * L, L)]
            canddata[w // 8, pl.ds((w % 8) * L, L)] = jnp.where(
                vid == 0, minus_inf, x)

        ovv = minus_inf
        oiv = zero_i
        for k in range(VK):
            macc = minus_inf
            for w in range(CAND_CAP):
                macc = jnp.maximum(
                    macc, canddata[w // 8, pl.ds((w % 8) * L, L)])
            m = jnp.max(macc)
            m_vec = jnp.full((L,), m, jnp.float32)

            iacc = big_vec
            for w in range(CAND_CAP):
                x = canddata[w // 8, pl.ds((w % 8) * L, L)]
                vid = vidbuf[pl.ds(w * L, L)]
                iacc = jnp.minimum(iacc, jnp.where(x == m_vec, vid, big_vec))
            ci = jnp.min(iacc)
            ci_vec = jnp.full((L,), ci, jnp.int32)

            for w in range(CAND_CAP):
                x = canddata[w // 8, pl.ds((w % 8) * L, L)]
                vid = vidbuf[pl.ds(w * L, L)]
                canddata[w // 8, pl.ds((w % 8) * L, L)] = jnp.where(
                    vid == ci_vec, minus_inf, x)

            ovv = jnp.where(lane == k, m_vec, ovv)
            oiv = jnp.where(lane == k, ci_vec, oiv)

        sh_vec = jnp.full((L,), shift, jnp.float32)
        st_val[pl.ds(rr * L, L)] = ovv + sh_vec
        st_idx[pl.ds(rr * L, L)] = oiv
        st_beam[pl.ds(rr * L, L)] = zero_i
        return row_carry

    with jax.named_scope("phaseB"):
        lax.fori_loop(0, ROWS_PER_TILE, row_body, 0)

    base = (c * (NS * ROWS_PER_TILE) + s * ROWS_PER_TILE) * L
    pltpu.sync_copy(st_val, out_val_hbm.at[pl.ds(base, ROWS_PER_TILE * L)])
    pltpu.sync_copy(st_idx, out_idx_hbm.at[pl.ds(base, ROWS_PER_TILE * L)])
    pltpu.sync_copy(st_beam, out_beam_hbm.at[pl.ds(base, ROWS_PER_TILE * L)])


@jax.jit
def _sc_topk(step_v, lp_flat, scores_flat):
    mesh = plsc.VectorSubcoreMesh(
        core_axis_name="c", subcore_axis_name="s",
        num_cores=NC, num_subcores=NS)
    fn = pl.kernel(
        _topk_body,
        out_type=(
            jax.ShapeDtypeStruct((BSZ * L,), jnp.float32),
            jax.ShapeDtypeStruct((BSZ * L,), jnp.int32),
            jax.ShapeDtypeStruct((BSZ * L,), jnp.int32),
        ),
        mesh=mesh,
        compiler_params=pltpu.CompilerParams(needs_layout_passes=False),
        scratch_types=[
            pltpu.VMEM((PIECE_W,), jnp.float32),
            pltpu.VMEM((PIECE_W,), jnp.float32),
            pltpu.VMEM((5 * CMC,), jnp.float32),
            pltpu.VMEM((CMC,), jnp.float32),
            pltpu.VMEM((BSZ * CMC,), jnp.float32),
            pltpu.VMEM((VB_CAP * CPB * CMC + CMC * L,),
                       jnp.float32),
            pltpu.VMEM((VB_CAP + 2 * L,), jnp.int32),
            pltpu.VMEM((CAND_CAP + L,), jnp.int32),
            pltpu.VMEM((CAND_CAP * L // BSZ, BSZ), jnp.int32),
            pltpu.VMEM((CAND_CAP * L // BSZ, BSZ), jnp.float32),
            pltpu.VMEM((CAND_CAP * L,), jnp.int32),
            pltpu.VMEM((ROWS_PER_TILE * L,), jnp.float32),
            pltpu.VMEM((ROWS_PER_TILE * L,), jnp.int32),
            pltpu.VMEM((ROWS_PER_TILE * L,), jnp.int32),
            pltpu.VMEM((L,), jnp.int32),
            pltpu.VMEM((BSZ * BEAMS,), jnp.float32),
            pltpu.VMEM_SHARED((NCHUNK * CMC,), jnp.float32),
            pltpu.VMEM_SHARED((NBLK * CMC,), jnp.float32),
            pltpu.SMEM((2,), jnp.int32),
            pltpu.SemaphoreType.DMA,
            pltpu.SemaphoreType.DMA,
        ],
    )
    return fn(step_v, lp_flat, scores_flat)


def kernel(step, lprobs, scores):
    step_v = jnp.broadcast_to(
        jnp.asarray(step, jnp.int32).reshape(()), (L,))
    lp_flat = jnp.transpose(lprobs, (1, 2, 0)).reshape(
        BEAMS, VOCAB // 8, 8, BSZ).reshape(-1)
    sc, ix, bm = _sc_topk(step_v, lp_flat, scores.reshape(-1))
    return (sc.reshape(BSZ, L)[:, :VK], ix.reshape(BSZ, L)[:, :VK],
            bm.reshape(BSZ, L)[:, :VK])


# speedup vs baseline: 1.5634x; 1.0257x over previous
import jax
import jax.numpy as jnp
from jax import lax
from jax.experimental import pallas as pl
from jax.experimental.pallas import tpu as pltpu
from jax.experimental.pallas import tpu_sc as plsc

NC = 2
NS = 16
L = 16

BSZ = 128
BEAMS = 4
VOCAB = 100000
VK = 8
ROWS_PER_TILE = BSZ // NC // NS
NCHUNK = VOCAB // L
CPB = 50
NBLK = NCHUNK // CPB
PIECE_V = 80
PIECE_W = PIECE_V * BSZ
PPB = 10
VB_CAP = 12
CAND_CAP = 32
CMC = BSZ // NC
GPS = CMC // L
CMS = 56

NEG_INF = float("-inf")
BIG_I32 = 2**31 - 1


def _topk_body(step_hbm, lp_hbm, scores_hbm,
               out_val_hbm, out_idx_hbm, out_beam_hbm,
               pbuf, pbuf2, cmstage, bmstage, bmloc, cmblk, blkids, cidlist,
               candidx, canddata, vidbuf, st_val, st_idx, st_beam,
               step_v, scores_v, cm_sh, bm_sh, cnt_smem, sem, sem2):
    c = lax.axis_index("c")
    s = lax.axis_index("s")
    lane = lax.iota(jnp.int32, L)
    minus_inf = jnp.full((L,), NEG_INF, jnp.float32)
    plus_inf = jnp.full((L,), float("inf"), jnp.float32)
    big_vec = jnp.full((L,), BIG_I32, jnp.int32)
    zero_i = jnp.zeros((L,), jnp.int32)

    pltpu.sync_copy(step_hbm, step_v)
    pltpu.sync_copy(scores_hbm, scores_v)
    ssum = jnp.zeros((L,), jnp.float32)
    for i in range(BSZ * BEAMS // L):
        ssum = ssum + scores_v[pl.ds(i * L, L)]
    mean = jnp.sum(ssum) * (1.0 / (BSZ * BEAMS))
    stepf = jnp.max(step_v[...].astype(jnp.float32))
    shift = stepf * mean

    sb = jnp.where(s < 13, 8 * s, 104 + 7 * (s - 13))
    nb = jnp.where(s < 13, 8, 7)

    np_ = nb * PPB

    def fire(p, buf, psem):
        vstart = sb * (CPB * L) + p * PIECE_V
        pltpu.async_copy(lp_hbm.at[pl.ds(vstart * BSZ, PIECE_W)], buf, psem)

    def wait_piece(buf, psem):
        pltpu.make_async_copy(lp_hbm.at[pl.ds(0, PIECE_W)], buf, psem).wait()

    def compute_piece(buf, p, bmacc):
        first = (s == 0) & (p == 0)
        pin = p % PPB
        new_acc = []
        for g in range(GPS):
            bacc_g = bmacc[g]
            for c5 in range(5):
                acc = minus_inf
                for j in range(L):
                    x = buf[pl.ds(
                        (c5 * L + j) * BSZ + (c * GPS + g) * L, L)]
                    if c5 == 0 and j == 0:
                        x = jnp.where(first, minus_inf, x)
                    acc = jnp.maximum(acc, x)
                plsc.store_scatter(
                    cmstage, [(g * L + lane) * CMS + pin * 5 + c5], acc)
                bacc_g = jnp.maximum(bacc_g, acc)
            new_acc.append(bacc_g)
        return new_acc

    with jax.named_scope("phaseA"):
        fire(0, pbuf, sem)
        fire(1, pbuf2, sem2)

        def pair_body(q, bmacc):
            p0 = 2 * q
            p1 = 2 * q + 1
            wait_piece(pbuf, sem)

            @pl.when(p0 + 2 < np_)
            def _f0():
                fire(p0 + 2, pbuf, sem)
            bmacc = tuple(compute_piece(pbuf, p0, bmacc))
            wait_piece(pbuf2, sem2)

            @pl.when(p1 + 2 < np_)
            def _f1():
                fire(p1 + 2, pbuf2, sem2)
            new_acc = compute_piece(pbuf2, p1, bmacc)

            @pl.when((p1 + 1) % PPB == 0)
            def _flush_bm():
                for g in range(GPS):
                    bmstage[pl.ds(g * L, L)] = new_acc[g]
                pltpu.sync_copy(
                    bmstage, bm_sh.at[pl.ds((sb + p1 // PPB) * CMC, CMC)])
                pltpu.sync_copy(
                    cmstage,
                    cm_sh.at[pl.ds((sb + p1 // PPB) * CMC * CMS, CMC * CMS)])

            done = (p1 + 1) % PPB == 0
            return tuple(jnp.where(done, minus_inf, a) for a in new_acc)

        lax.fori_loop(0, np_ // 2, pair_body, (minus_inf,) * GPS)

    plsc.subcore_barrier()

    pltpu.sync_copy(bm_sh, bmloc.at[pl.ds(0, NBLK * CMC)])

    def row_body(rr, row_carry):
        rloc = s * ROWS_PER_TILE + rr
        r = c * (NS * ROWS_PER_TILE) + rloc

        bmo = []
        for k in range(8):
            bv = plsc.load_gather(bmloc, [(lane + k * L) * CMC + rloc])
            if k == 7:
                bv = jnp.where(lane + 112 < NBLK, bv, minus_inf)
            bmo.append(bv)

        wk = list(bmo)
        t = jnp.float32(0)
        for it in range(VK):
            mx = wk[0]
            for k in range(1, 8):
                mx = jnp.maximum(mx, wk[k])
            m = jnp.max(mx)
            m_vec = jnp.full((L,), m, jnp.float32)
            wk = [jnp.where(w == m_vec, minus_inf, w) for w in wk]
            t = m
        t_vec = jnp.full((L,), t, jnp.float32)

        cnt_smem[0] = 0
        for k in range(8):
            hits = bmo[k] >= t_vec
            s0 = cnt_smem[0]

            @pl.when(s0 <= VB_CAP)
            def _stb(hits=hits, k=k, s0=s0):
                plsc.store_compressed(
                    blkids.at[pl.ds(s0, L)], lane + k * L, mask=hits)
                cnt_smem[0] = s0 + jnp.max(
                    plsc.all_reduce_population_count(hits))
        nvb = cnt_smem[0]
        bv16 = blkids[pl.ds(0, L)]

        for j in range(VB_CAP):
            b = bv16[j]

            @pl.when(j < nvb)
            def _fire(b=b, j=j):
                pltpu.async_copy(
                    cm_sh.at[pl.ds((b * CMC + rloc) * CMS, CMS)],
                    cmblk.at[pl.ds(j * 4 * L, CMS)], sem)
        for j in range(VB_CAP):
            @pl.when(j < nvb)
            def _drain(j=j):
                pltpu.make_async_copy(
                    lp_hbm.at[pl.ds(0, CMS)],
                    cmblk.at[pl.ds(j * 4 * L, CMS)], sem).wait()

        cnt_smem[1] = 0
        for j in range(VB_CAP):
            b = bv16[j]

            @pl.when(j < nvb)
            def _cand(b=b, j=j):
                for k4 in range(4):
                    cloc = lane + k4 * L
                    col = cmblk[pl.ds(j * 4 * L + k4 * L, L)]
                    h = (col >= t_vec) & (cloc < CPB)
                    s2 = cnt_smem[1]

                    @pl.when(s2 <= CAND_CAP - L)
                    def _stc(h=h, b=b, cloc=cloc, s2=s2):
                        plsc.store_compressed(
                            cidlist.at[pl.ds(s2, L)], b * CPB + cloc, mask=h)
                        cnt_smem[1] = s2 + jnp.max(
                            plsc.all_reduce_population_count(h))
        ncand = cnt_smem[1]
        cidlist[pl.ds(ncand, L)] = zero_i

        ncand_vec = jnp.full((L,), ncand, jnp.int32)
        for q in range(CAND_CAP // L):
            cidv = cidlist[pl.ds(q * L, L)]
            cidv = jnp.where(lane + q * L < ncand_vec, cidv, 0)
            for j16 in range(L):
                v = cidv * L + j16
                addr = (v // 8) * (8 * BSZ) + (v % 8) * BSZ + r
                w = q * L + j16
                candidx[w // 8, pl.ds((w % 8) * L, L)] = addr
                vidbuf[pl.ds(w * L, L)] = v
        for q2 in range(CAND_CAP * L // BSZ):
            pltpu.async_copy(
                lp_hbm.at[candidx.at[q2]], canddata.at[q2], sem2)
        for q2 in range(CAND_CAP * L // BSZ):
            pltpu.make_async_copy(
                lp_hbm.at[pl.ds(0, BSZ)], canddata.at[q2], sem2).wait()

        for w in range(CAND_CAP):
            x = canddata[w // 8, pl.ds((w % 8) * L, L)]
            vid = vidbuf[pl.ds(w * L, L)]
            canddata[w // 8, pl.ds((w % 8) * L, L)] = jnp.where(
                vid == 0, minus_inf, x)

        ovv = minus_inf
        oiv = zero_i
        for k in range(VK):
            macc = minus_inf
            for w in range(CAND_CAP):
                macc = jnp.maximum(
                    macc, canddata[w // 8, pl.ds((w % 8) * L, L)])
            m = jnp.max(macc)
            m_vec = jnp.full((L,), m, jnp.float32)

            iacc = big_vec
            for w in range(CAND_CAP):
                x = canddata[w // 8, pl.ds((w % 8) * L, L)]
                vid = vidbuf[pl.ds(w * L, L)]
                iacc = jnp.minimum(iacc, jnp.where(x == m_vec, vid, big_vec))
            ci = jnp.min(iacc)
            ci_vec = jnp.full((L,), ci, jnp.int32)

            for w in range(CAND_CAP):
                x = canddata[w // 8, pl.ds((w % 8) * L, L)]
                vid = vidbuf[pl.ds(w * L, L)]
                canddata[w // 8, pl.ds((w % 8) * L, L)] = jnp.where(
                    vid == ci_vec, minus_inf, x)

            ovv = jnp.where(lane == k, m_vec, ovv)
            oiv = jnp.where(lane == k, ci_vec, oiv)

        sh_vec = jnp.full((L,), shift, jnp.float32)
        st_val[pl.ds(rr * L, L)] = ovv + sh_vec
        st_idx[pl.ds(rr * L, L)] = oiv
        st_beam[pl.ds(rr * L, L)] = zero_i
        return row_carry

    with jax.named_scope("phaseB"):
        lax.fori_loop(0, ROWS_PER_TILE, row_body, 0)

    base = (c * (NS * ROWS_PER_TILE) + s * ROWS_PER_TILE) * L
    pltpu.sync_copy(st_val, out_val_hbm.at[pl.ds(base, ROWS_PER_TILE * L)])
    pltpu.sync_copy(st_idx, out_idx_hbm.at[pl.ds(base, ROWS_PER_TILE * L)])
    pltpu.sync_copy(st_beam, out_beam_hbm.at[pl.ds(base, ROWS_PER_TILE * L)])


@jax.jit
def _sc_topk(step_v, lp_flat, scores_flat):
    mesh = plsc.VectorSubcoreMesh(
        core_axis_name="c", subcore_axis_name="s",
        num_cores=NC, num_subcores=NS)
    fn = pl.kernel(
        _topk_body,
        out_type=(
            jax.ShapeDtypeStruct((BSZ * L,), jnp.float32),
            jax.ShapeDtypeStruct((BSZ * L,), jnp.int32),
            jax.ShapeDtypeStruct((BSZ * L,), jnp.int32),
        ),
        mesh=mesh,
        compiler_params=pltpu.CompilerParams(needs_layout_passes=False),
        scratch_types=[
            pltpu.VMEM((PIECE_W,), jnp.float32),
            pltpu.VMEM((PIECE_W,), jnp.float32),
            pltpu.VMEM((CMC * CMS,), jnp.float32),
            pltpu.VMEM((CMC,), jnp.float32),
            pltpu.VMEM((BSZ * CMC,), jnp.float32),
            pltpu.VMEM((VB_CAP * 4 * L,), jnp.float32),
            pltpu.VMEM((VB_CAP + 2 * L,), jnp.int32),
            pltpu.VMEM((CAND_CAP + L,), jnp.int32),
            pltpu.VMEM((CAND_CAP * L // BSZ, BSZ), jnp.int32),
            pltpu.VMEM((CAND_CAP * L // BSZ, BSZ), jnp.float32),
            pltpu.VMEM((CAND_CAP * L,), jnp.int32),
            pltpu.VMEM((ROWS_PER_TILE * L,), jnp.float32),
            pltpu.VMEM((ROWS_PER_TILE * L,), jnp.int32),
            pltpu.VMEM((ROWS_PER_TILE * L,), jnp.int32),
            pltpu.VMEM((L,), jnp.int32),
            pltpu.VMEM((BSZ * BEAMS,), jnp.float32),
            pltpu.VMEM_SHARED((NBLK * CMC * CMS,), jnp.float32),
            pltpu.VMEM_SHARED((NBLK * CMC,), jnp.float32),
            pltpu.SMEM((2,), jnp.int32),
            pltpu.SemaphoreType.DMA,
            pltpu.SemaphoreType.DMA,
        ],
    )
    return fn(step_v, lp_flat, scores_flat)


def kernel(step, lprobs, scores):
    step_v = jnp.broadcast_to(
        jnp.asarray(step, jnp.int32).reshape(()), (L,))
    lp_flat = jnp.transpose(lprobs, (1, 2, 0)).reshape(
        BEAMS, VOCAB // 8, 8, BSZ).reshape(-1)
    sc, ix, bm = _sc_topk(step_v, lp_flat, scores.reshape(-1))
    return (sc.reshape(BSZ, L)[:, :VK], ix.reshape(BSZ, L)[:, :VK],
            bm.reshape(BSZ, L)[:, :VK])
